# Initial kernel scaffold; baseline (speedup 1.0000x reference)
import jax
import jax.numpy as jnp
from jax.experimental import pallas as pl


def kernel(x, edge_index, Wl0, bl0, Wr0, Wl1, bl1, Wr1, Wl2, bl2, Wr2, Wl3, bl3, Wr3):
    raise NotImplementedError("write your pallas kernel here")


# speedup vs baseline: 9.9205x; 9.9205x over previous
import jax
import jax.numpy as jnp
from jax import lax
from jax.experimental import pallas as pl
from jax.experimental.pallas import tpu as pltpu
from jax.experimental.pallas import tpu_sc as plsc

N = 10000
E = 320000
D = 128
NC = 2
NS = 16
NB = 125
NR = E // NB
NIT = NR // (NC * NS)
NPH = 2
NITP = NIT // NPH
NIT_C = NR // NS
RPT = N // NS
BR = 2000

def _mesh():
    return plsc.VectorSubcoreMesh(core_axis_name="c", subcore_axis_name="s",
                                  num_cores=NC, num_subcores=NS)


def _agg_body(g_hbm, src_hbm, dst_hbm, zeros_hbm, out_hbm,
              acc, srcv, dstv, rows0, rows1, sem0, sem1):
    c = lax.axis_index("c").astype(jnp.int32)
    s = lax.axis_index("s").astype(jnp.int32)
    i32 = jnp.int32
    pltpu.sync_copy(zeros_hbm, acc.at[pl.ds(s * i32(RPT), RPT)])
    plsc.subcore_barrier()
    row0 = (c * i32(NS) + s) * i32(NIT)

    for p in range(NPH):
        rp = row0 + i32(p * NITP)
        pltpu.sync_copy(src_hbm.at[pl.ds(rp, NITP)], srcv)
        pltpu.sync_copy(dst_hbm.at[pl.ds(rp, NITP)], dstv)

        pltpu.async_copy(g_hbm.at[srcv.at[i32(0)]], rows0, sem0)

        @pl.loop(i32(0), i32(NITP), step=i32(2))
        def _(j):
            j0 = j.astype(jnp.int32)
            j1 = j0 + i32(1)
            j2 = j0 + i32(2)
            pltpu.async_copy(g_hbm.at[srcv.at[j1]], rows1, sem1)
            pltpu.make_async_copy(g_hbm.at[srcv.at[j0]], rows0, sem0).wait()
            pltpu.sync_copy(rows0, acc.at[dstv.at[j0]], add=True)

            @pl.when(j2 < i32(NITP))
            def _():
                pltpu.async_copy(g_hbm.at[srcv.at[j2]], rows0, sem0)

            pltpu.make_async_copy(g_hbm.at[srcv.at[j1]], rows1, sem1).wait()
            pltpu.sync_copy(rows1, acc.at[dstv.at[j1]], add=True)

    plsc.subcore_barrier()
    pltpu.sync_copy(acc.at[pl.ds(s * i32(RPT), RPT)], out_hbm.at[c, s])


def _segment_sum(g, src2d, dst2d, zeros):
    return pl.kernel(
        _agg_body,
        out_type=jax.ShapeDtypeStruct((NC, NS, RPT, D), jnp.float32),
        mesh=_mesh(),
        scratch_types=[
            pltpu.VMEM_SHARED((N, D), jnp.float32),
            pltpu.VMEM((NITP, NB), jnp.int32),
            pltpu.VMEM((NITP, NB), jnp.int32),
            pltpu.VMEM((NB, D), jnp.float32),
            pltpu.VMEM((NB, D), jnp.float32),
            pltpu.SemaphoreType.DMA,
            pltpu.SemaphoreType.DMA,
        ],
    )(g, src2d, dst2d, zeros)


def _counts_body(dst_hbm, ones_hbm, zeros_hbm, out_hbm, acc, dstv, onesv):
    c = lax.axis_index("c").astype(jnp.int32)
    s = lax.axis_index("s").astype(jnp.int32)
    i32 = jnp.int32
    pltpu.sync_copy(zeros_hbm, acc.at[pl.ds(s * i32(RPT), RPT)])
    pltpu.sync_copy(ones_hbm, onesv)
    row0 = (c * i32(NS) + s) * i32(NIT)
    pltpu.sync_copy(dst_hbm.at[pl.ds(row0, NIT)], dstv)
    plsc.subcore_barrier()

    @pl.loop(i32(0), i32(NIT))
    def _(j):
        pltpu.sync_copy(onesv, acc.at[dstv.at[j.astype(jnp.int32)]], add=True)

    plsc.subcore_barrier()
    pltpu.sync_copy(acc.at[pl.ds(s * i32(RPT), RPT)], out_hbm.at[c, s])


def _counts(dst2d, ones2d, zeros):
    return pl.kernel(
        _counts_body,
        out_type=jax.ShapeDtypeStruct((NC, NS, RPT, D), jnp.float32),
        mesh=_mesh(),
        scratch_types=[
            pltpu.VMEM_SHARED((N, D), jnp.float32),
            pltpu.VMEM((NIT, NB), jnp.int32),
            pltpu.VMEM((NB, D), jnp.float32),
        ],
    )(dst2d, ones2d, zeros)


def _zi(i):
    return jnp.zeros_like(i)


def _dot_t(a, w):
    return lax.dot_general(a, w, (((1,), (1,)), ((), ())),
                           preferred_element_type=jnp.float32)


def _dense0_body(x_ref, wl_ref, wr_ref, bl_ref, g_ref, r_ref):
    xb = x_ref[...]
    g_ref[...] = _dot_t(xb, wl_ref[...])
    r_ref[...] = _dot_t(xb, wr_ref[...]) + bl_ref[...]


def _dense0(x, wl, wr, bl2d):
    return pl.pallas_call(
        _dense0_body,
        grid=(N // BR,),
        in_specs=[
            pl.BlockSpec((BR, D), lambda i: (i, _zi(i))),
            pl.BlockSpec((D, D), lambda i: (_zi(i), _zi(i))),
            pl.BlockSpec((D, D), lambda i: (_zi(i), _zi(i))),
            pl.BlockSpec((1, D), lambda i: (_zi(i), _zi(i))),
        ],
        out_specs=[
            pl.BlockSpec((BR, D), lambda i: (i, _zi(i))),
            pl.BlockSpec((BR, D), lambda i: (i, _zi(i))),
        ],
        out_shape=[
            jax.ShapeDtypeStruct((N, D), jnp.float32),
            jax.ShapeDtypeStruct((N, D), jnp.float32),
        ],
    )(x, wl, wr, bl2d)


def _invert_body(c_ref, o_ref):
    o_ref[...] = 1.0 / jnp.maximum(c_ref[0] + c_ref[1], 1.0)


def _invert(cp):
    return pl.pallas_call(
        _invert_body,
        grid=(N // BR,),
        in_specs=[pl.BlockSpec((NC, BR, D), lambda i: (_zi(i), i, _zi(i)))],
        out_specs=pl.BlockSpec((BR, D), lambda i: (i, _zi(i))),
        out_shape=jax.ShapeDtypeStruct((N, D), jnp.float32),
    )(cp)


def _fused_body(s_ref, inv_ref, r_ref, wl_ref, wr_ref, bl_ref, g_ref, r_out):
    h = jax.nn.relu((s_ref[0] + s_ref[1]) * inv_ref[...] + r_ref[...])
    g_ref[...] = _dot_t(h, wl_ref[...])
    r_out[...] = _dot_t(h, wr_ref[...]) + bl_ref[...]


def _fused(sp, inv2d, r, wl, wr, bl2d):
    return pl.pallas_call(
        _fused_body,
        grid=(N // BR,),
        in_specs=[
            pl.BlockSpec((NC, BR, D), lambda i: (_zi(i), i, _zi(i))),
            pl.BlockSpec((BR, D), lambda i: (i, _zi(i))),
            pl.BlockSpec((BR, D), lambda i: (i, _zi(i))),
            pl.BlockSpec((D, D), lambda i: (_zi(i), _zi(i))),
            pl.BlockSpec((D, D), lambda i: (_zi(i), _zi(i))),
            pl.BlockSpec((1, D), lambda i: (_zi(i), _zi(i))),
        ],
        out_specs=[
            pl.BlockSpec((BR, D), lambda i: (i, _zi(i))),
            pl.BlockSpec((BR, D), lambda i: (i, _zi(i))),
        ],
        out_shape=[
            jax.ShapeDtypeStruct((N, D), jnp.float32),
            jax.ShapeDtypeStruct((N, D), jnp.float32),
        ],
    )(sp, inv2d, r, wl, wr, bl2d)


def _epilogue_body(s_ref, inv_ref, r_ref, o_ref):
    o_ref[...] = jax.nn.relu((s_ref[0] + s_ref[1]) * inv_ref[...] + r_ref[...])


def _epilogue(sp, inv2d, r):
    return pl.pallas_call(
        _epilogue_body,
        grid=(N // BR,),
        in_specs=[
            pl.BlockSpec((NC, BR, D), lambda i: (_zi(i), i, _zi(i))),
            pl.BlockSpec((BR, D), lambda i: (i, _zi(i))),
            pl.BlockSpec((BR, D), lambda i: (i, _zi(i))),
        ],
        out_specs=pl.BlockSpec((BR, D), lambda i: (i, _zi(i))),
        out_shape=jax.ShapeDtypeStruct((N, D), jnp.float32),
    )(sp, inv2d, r)


def kernel(x, edge_index, Wl0, bl0, Wr0, Wl1, bl1, Wr1, Wl2, bl2, Wr2,
           Wl3, bl3, Wr3):
    x = x.astype(jnp.float32)
    src2d = edge_index[0].astype(jnp.int32).reshape(NR, NB)
    dst2d = edge_index[1].astype(jnp.int32).reshape(NR, NB)
    zeros = jnp.zeros((RPT, D), jnp.float32)
    ones2d = jnp.ones((NB, D), jnp.float32)
    params = [(Wl0, bl0, Wr0), (Wl1, bl1, Wr1), (Wl2, bl2, Wr2),
              (Wl3, bl3, Wr3)]

    cp = _counts(dst2d, ones2d, zeros).reshape(NC, N, D)
    inv2d = _invert(cp)
    g, r = _dense0(x, params[0][0], params[0][2],
                   params[0][1].reshape(1, D).astype(jnp.float32))
    for wl, bl, wr in params[1:]:
        sp = _segment_sum(g, src2d, dst2d, zeros).reshape(NC, N, D)
        g, r = _fused(sp, inv2d, r, wl, wr, bl.reshape(1, D))
    sp = _segment_sum(g, src2d, dst2d, zeros).reshape(NC, N, D)
    return _epilogue(sp, inv2d, r)
